# baseline (speedup 1.0000x reference)
import jax
import jax.numpy as jnp
from jax.experimental import pallas as pl
from jax.experimental.pallas import tpu as pltpu

_NV = 8
_EPS = 1e-5
_TT = 128


def _fused_body(x_ref, ln_g_ref, ln_b_ref, W1_ref, b1_ref, W2_ref, b2_ref,
                gW_ref, gb_ref, out_ref):
    C = x_ref.shape[1]
    T = x_ref.shape[2]
    ln_g = ln_g_ref[...]
    ln_b = ln_b_ref[...]
    gb = gb_ref[...]
    W1 = W1_ref[...]
    W2 = W2_ref[...]

    gates = []
    for v in range(_NV):
        pooled_v = jnp.mean(x_ref[v], axis=1, keepdims=True)
        logit_v = jax.lax.dot_general(
            gW_ref[...], pooled_v, (((1,), (0,)), ((), ())),
            preferred_element_type=jnp.float32) + gb
        gates.append(jax.nn.sigmoid(logit_v))

    for ts in range(0, T, _TT):
        tsl = slice(ts, ts + _TT)
        wsum = None
        corr = None
        for v in range(_NV):
            xv = x_ref[v, :, tsl]
            mu_v = jnp.mean(xv, axis=0, keepdims=True)
            msq_v = jnp.mean(xv * xv, axis=0, keepdims=True)
            r_v = jax.lax.rsqrt(msq_v - mu_v * mu_v + _EPS)
            term = xv * r_v
            cterm = mu_v * r_v
            wsum = term if wsum is None else wsum + term
            corr = cterm if corr is None else corr + cterm

        s = ln_g * ((wsum - corr) * (1.0 / _NV)) + ln_b

        h1 = jax.lax.dot_general(
            W1, s, (((0,), (0,)), ((), ())),
            preferred_element_type=jnp.float32) + b1_ref[...]

        mu2 = jnp.mean(h1, axis=0, keepdims=True)
        var2 = jnp.mean((h1 - mu2) ** 2, axis=0, keepdims=True)
        a = jnp.maximum((h1 - mu2) * jax.lax.rsqrt(var2 + _EPS), 0.0)

        h2 = jax.lax.dot_general(
            W2, a, (((0,), (0,)), ((), ())),
            preferred_element_type=jnp.float32) + b2_ref[...]

        for v in range(_NV):
            out_ref[v, :, tsl] = x_ref[v, :, tsl] + gates[v] * h2


@jax.jit
def kernel(x, data_key, ln_g, ln_b, W1, b1, W2, b2, gate_W, gate_b):
    B, C, T = x.shape
    n_groups = B // _NV

    in_specs = [
        pl.BlockSpec((_NV, C, T), lambda i: (i, 0, 0)),
        pl.BlockSpec((C, 1), lambda i: (0, 0)),
        pl.BlockSpec((C, 1), lambda i: (0, 0)),
        pl.BlockSpec((C, C), lambda i: (0, 0)),
        pl.BlockSpec((C, 1), lambda i: (0, 0)),
        pl.BlockSpec((C, C), lambda i: (0, 0)),
        pl.BlockSpec((C, 1), lambda i: (0, 0)),
        pl.BlockSpec((C, C), lambda i: (0, 0)),
        pl.BlockSpec((C, 1), lambda i: (0, 0)),
    ]

    return pl.pallas_call(
        _fused_body,
        grid=(n_groups,),
        in_specs=in_specs,
        out_specs=pl.BlockSpec((_NV, C, T), lambda i: (i, 0, 0)),
        out_shape=jax.ShapeDtypeStruct((B, C, T), x.dtype),
        compiler_params=pltpu.CompilerParams(
            dimension_semantics=("arbitrary",),
        ),
    )(x, ln_g.reshape(C, 1), ln_b.reshape(C, 1), W1, b1.reshape(C, 1),
      W2, b2.reshape(C, 1), gate_W, gate_b.reshape(C, 1))


# speedup vs baseline: 1.0344x; 1.0344x over previous
import jax
import jax.numpy as jnp
from jax.experimental import pallas as pl
from jax.experimental.pallas import tpu as pltpu

_NV = 8
_EPS = 1e-5


def _inner(ln_g_ref, ln_b_ref, W1_ref, b1_ref, W2_ref, b2_ref,
           gW_ref, gb_ref, x_ref, out_ref):
    ln_g = ln_g_ref[...]
    ln_b = ln_b_ref[...]
    gb = gb_ref[...]

    wsum = None
    corr = None
    gates = []
    for v in range(_NV):
        xv = x_ref[v]
        mu_v = jnp.mean(xv, axis=0, keepdims=True)
        msq_v = jnp.mean(xv * xv, axis=0, keepdims=True)
        r_v = jax.lax.rsqrt(msq_v - mu_v * mu_v + _EPS)
        term = xv * r_v
        cterm = mu_v * r_v
        wsum = term if wsum is None else wsum + term
        corr = cterm if corr is None else corr + cterm
        pooled_v = jnp.mean(xv, axis=1, keepdims=True)
        logit_v = jax.lax.dot_general(
            gW_ref[...], pooled_v, (((1,), (0,)), ((), ())),
            preferred_element_type=jnp.float32) + gb
        gates.append(jax.nn.sigmoid(logit_v))

    s = ln_g * ((wsum - corr) * (1.0 / _NV)) + ln_b

    h1 = jax.lax.dot_general(
        W1_ref[...], s, (((0,), (0,)), ((), ())),
        preferred_element_type=jnp.float32) + b1_ref[...]

    mu2 = jnp.mean(h1, axis=0, keepdims=True)
    var2 = jnp.mean((h1 - mu2) ** 2, axis=0, keepdims=True)
    a = jnp.maximum((h1 - mu2) * jax.lax.rsqrt(var2 + _EPS), 0.0)

    h2 = jax.lax.dot_general(
        W2_ref[...], a, (((0,), (0,)), ((), ())),
        preferred_element_type=jnp.float32) + b2_ref[...]

    for v in range(_NV):
        out_ref[v] = x_ref[v] + gates[v] * h2


def _outer(x_hbm, ln_g_ref, ln_b_ref, W1_ref, b1_ref, W2_ref, b2_ref,
           gW_ref, gb_ref, out_hbm):
    n_groups = x_hbm.shape[0] // _NV
    C = x_hbm.shape[1]
    T = x_hbm.shape[2]
    pltpu.emit_pipeline(
        lambda xr, orr: _inner(ln_g_ref, ln_b_ref, W1_ref, b1_ref, W2_ref,
                               b2_ref, gW_ref, gb_ref, xr, orr),
        grid=(n_groups,),
        in_specs=[pl.BlockSpec((_NV, C, T), lambda i: (i, 0, 0))],
        out_specs=[pl.BlockSpec((_NV, C, T), lambda i: (i, 0, 0))],
    )(x_hbm, out_hbm)


@jax.jit
def kernel(x, data_key, ln_g, ln_b, W1, b1, W2, b2, gate_W, gate_b):
    B, C, T = x.shape

    in_specs = [
        pl.BlockSpec(memory_space=pl.ANY),
        pl.BlockSpec((C, 1), lambda: (0, 0)),
        pl.BlockSpec((C, 1), lambda: (0, 0)),
        pl.BlockSpec((C, C), lambda: (0, 0)),
        pl.BlockSpec((C, 1), lambda: (0, 0)),
        pl.BlockSpec((C, C), lambda: (0, 0)),
        pl.BlockSpec((C, 1), lambda: (0, 0)),
        pl.BlockSpec((C, C), lambda: (0, 0)),
        pl.BlockSpec((C, 1), lambda: (0, 0)),
    ]

    return pl.pallas_call(
        _outer,
        in_specs=in_specs,
        out_specs=pl.BlockSpec(memory_space=pl.ANY),
        out_shape=jax.ShapeDtypeStruct((B, C, T), x.dtype),
    )(x, ln_g.reshape(C, 1), ln_b.reshape(C, 1), W1, b1.reshape(C, 1),
      W2, b2.reshape(C, 1), gate_W, gate_b.reshape(C, 1))


# speedup vs baseline: 1.0654x; 1.0300x over previous
import functools

import jax
import jax.numpy as jnp
from jax.experimental import pallas as pl
from jax.experimental.pallas import tpu as pltpu

_NV = 8
_EPS = 1e-5


def _fused_body(x_ref, ln_g_ref, ln_b_ref, W1_ref, b1_ref, W2_ref, b2_ref,
                gW_ref, gb_ref, out_ref):
    xb = x_ref[...]

    mu = jnp.mean(xb, axis=1)
    msq = jnp.mean(xb * xb, axis=1)
    r = jax.lax.rsqrt(msq - mu * mu + _EPS)
    wsum = jnp.sum(xb * r[:, None, :], axis=0)
    corr = jnp.mean(mu * r, axis=0, keepdims=True)
    s = ln_g_ref[...] * (wsum * (1.0 / _NV) - corr) + ln_b_ref[...]

    h1 = jax.lax.dot_general(
        W1_ref[...], s, (((0,), (0,)), ((), ())),
        preferred_element_type=jnp.float32,
        precision=jax.lax.Precision.DEFAULT) + b1_ref[...]

    mu2 = jnp.mean(h1, axis=0, keepdims=True)
    var2 = jnp.mean((h1 - mu2) ** 2, axis=0, keepdims=True)
    a = jnp.maximum((h1 - mu2) * jax.lax.rsqrt(var2 + _EPS), 0.0)

    h2 = jax.lax.dot_general(
        W2_ref[...], a, (((0,), (0,)), ((), ())),
        preferred_element_type=jnp.float32,
        precision=jax.lax.Precision.DEFAULT) + b2_ref[...]

    pooled = jnp.mean(xb, axis=2)
    logits = jax.lax.dot_general(
        pooled, gW_ref[...], (((1,), (1,)), ((), ())),
        preferred_element_type=jnp.float32,
        precision=jax.lax.Precision.DEFAULT) + gb_ref[...]
    gate = jax.nn.sigmoid(logits)

    out_ref[...] = xb + gate[:, :, None] * h2[None, :, :]


@jax.jit
def kernel(x, data_key, ln_g, ln_b, W1, b1, W2, b2, gate_W, gate_b):
    B, C, T = x.shape
    n_groups = B // _NV

    grid_spec = pl.GridSpec(
        grid=(n_groups,),
        in_specs=[
            pl.BlockSpec((_NV, C, T), lambda i: (i, 0, 0)),
            pl.BlockSpec((C, 1), lambda i: (0, 0)),
            pl.BlockSpec((C, 1), lambda i: (0, 0)),
            pl.BlockSpec((C, C), lambda i: (0, 0)),
            pl.BlockSpec((C, 1), lambda i: (0, 0)),
            pl.BlockSpec((C, C), lambda i: (0, 0)),
            pl.BlockSpec((C, 1), lambda i: (0, 0)),
            pl.BlockSpec((C, C), lambda i: (0, 0)),
            pl.BlockSpec((1, C), lambda i: (0, 0)),
        ],
        out_specs=pl.BlockSpec((_NV, C, T), lambda i: (i, 0, 0)),
    )

    return pl.pallas_call(
        _fused_body,
        grid_spec=grid_spec,
        out_shape=jax.ShapeDtypeStruct((B, C, T), x.dtype),
        compiler_params=pltpu.CompilerParams(
            dimension_semantics=("parallel",),
        ),
    )(x, ln_g.reshape(C, 1), ln_b.reshape(C, 1), W1, b1.reshape(C, 1),
      W2, b2.reshape(C, 1), gate_W, gate_b.reshape(1, C))
